# baseline (speedup 1.0000x reference)
import functools

import numpy as np
import jax
import jax.numpy as jnp
from jax import lax
from jax.experimental import pallas as pl
from jax.experimental.pallas import tpu as pltpu
from jax.experimental.pallas import tpu_sc as plsc

N_ATOMS = 10000
N_BONDS = 160000
MAX_NB = 16
ATOM_FDIM = 128
BOND_FDIM = 144
HIDDEN = 256
N_MOLS = 100
PRIOR_SIG = 0.1
DEPTH_MAX = 5

NA_PAD = 10240
L = 16

_info = plsc.get_sparse_core_info()
_NC, _NS = _info.num_cores, _info.num_subcores
NW = _NC * _NS

_BM = 1600


def _pack_msg(x):
    xb = x.astype(jnp.bfloat16)
    lo = lax.bitcast_convert_type(xb[:, :128], jnp.uint16).astype(jnp.int32)
    hi = lax.bitcast_convert_type(xb[:, 128:], jnp.uint16).astype(jnp.int32)
    return lo | (hi << 16)


def _unpack_msg_bf(w):
    lo = lax.bitcast_convert_type((w & 0xFFFF).astype(jnp.uint16), jnp.bfloat16)
    hi = lax.bitcast_convert_type(
        lax.shift_right_logical(w, 16).astype(jnp.uint16), jnp.bfloat16)
    return jnp.concatenate([lo, hi], axis=1)


def _wi_body(x_ref, w_ref, inp_ref, msg_ref, msgb_ref, msgc_ref, msgd_ref):
    acc = jnp.dot(x_ref[...], w_ref[...], preferred_element_type=jnp.float32)
    inp_ref[...] = _pack_msg(acc)
    m = _pack_msg(jnp.maximum(acc, 0.0))
    msg_ref[...] = m
    msgb_ref[...] = m
    msgc_ref[...] = m
    msgd_ref[...] = m


def _wi_call(f_bonds, wi_t):
    grid = N_BONDS // _BM
    return pl.pallas_call(
        _wi_body,
        grid=(grid,),
        in_specs=[
            pl.BlockSpec((_BM, BOND_FDIM), lambda t: (t, 0)),
            pl.BlockSpec((BOND_FDIM, HIDDEN), lambda t: (0, 0)),
        ],
        out_specs=[pl.BlockSpec((_BM, 128), lambda t: (t, 0))] * 5,
        out_shape=[jax.ShapeDtypeStruct((N_BONDS, 128), jnp.int32)] * 5,
    )(f_bonds, wi_t)


def _mm_body(m_ref, w_ref, h_ref, hb_ref):
    m = _unpack_msg_bf(m_ref[...])
    acc = jnp.dot(m, w_ref[...], preferred_element_type=jnp.float32)
    hp = _pack_msg(acc)
    h_ref[...] = hp
    hb_ref[...] = hp


def _mm_call(msg3, wh_bf):
    grid = N_BONDS // _BM
    return pl.pallas_call(
        _mm_body,
        grid=(grid,),
        in_specs=[
            pl.BlockSpec((_BM, 128), lambda t: (t, 0)),
            pl.BlockSpec((HIDDEN, HIDDEN), lambda t: (0, 0)),
        ],
        out_specs=[pl.BlockSpec((_BM, 128), lambda t: (t, 0))] * 2,
        out_shape=[jax.ShapeDtypeStruct((N_BONDS, 128), jnp.int32)] * 2,
    )(msg3, wh_bf)


def _relu_mm_body(ga_ref, gh_ref, inp_ref, w_ref, msg_ref, msgb_ref, msgc_ref, msgd_ref, h_ref, hb_ref):
    a = _unpack_msg_bf(ga_ref[...]).astype(jnp.float32)
    h = _unpack_msg_bf(gh_ref[...]).astype(jnp.float32)
    i = _unpack_msg_bf(inp_ref[...]).astype(jnp.float32)
    m = jnp.maximum(i + (a - h), 0.0)
    mp = _pack_msg(m)
    msg_ref[...] = mp
    msgb_ref[...] = mp
    msgc_ref[...] = mp
    msgd_ref[...] = mp
    acc = jnp.dot(m.astype(jnp.bfloat16), w_ref[...],
                  preferred_element_type=jnp.float32)
    hp = _pack_msg(acc)
    h_ref[...] = hp
    hb_ref[...] = hp


def _relu_mm_call(ga, gh, inp, wh_bf):
    grid = N_BONDS // _BM
    return pl.pallas_call(
        _relu_mm_body,
        grid=(grid,),
        in_specs=[
            pl.BlockSpec((_BM, 128), lambda t: (t, 0)),
            pl.BlockSpec((_BM, 128), lambda t: (t, 0)),
            pl.BlockSpec((_BM, 128), lambda t: (t, 0)),
            pl.BlockSpec((HIDDEN, HIDDEN), lambda t: (0, 0)),
        ],
        out_specs=[pl.BlockSpec((_BM, 128), lambda t: (t, 0))] * 6,
        out_shape=[jax.ShapeDtypeStruct((N_BONDS, 128), jnp.int32)] * 6,
    )(ga, gh, inp, wh_bf)


def _relu_body(ga_ref, gh_ref, inp_ref, msg_ref, msgb_ref, msgc_ref, msgd_ref):
    a = _unpack_msg_bf(ga_ref[...]).astype(jnp.float32)
    h = _unpack_msg_bf(gh_ref[...]).astype(jnp.float32)
    i = _unpack_msg_bf(inp_ref[...]).astype(jnp.float32)
    m = _pack_msg(jnp.maximum(i + (a - h), 0.0))
    msg_ref[...] = m
    msgb_ref[...] = m
    msgc_ref[...] = m
    msgd_ref[...] = m


def _relu_call(ga, gh, inp):
    grid = N_BONDS // _BM
    return pl.pallas_call(
        _relu_body,
        grid=(grid,),
        in_specs=[
            pl.BlockSpec((_BM, 128), lambda t: (t, 0)),
            pl.BlockSpec((_BM, 128), lambda t: (t, 0)),
            pl.BlockSpec((_BM, 128), lambda t: (t, 0)),
        ],
        out_specs=[pl.BlockSpec((_BM, 128), lambda t: (t, 0))] * 4,
        out_shape=[jax.ShapeDtypeStruct((N_BONDS, 128), jnp.int32)] * 4,
    )(ga, gh, inp)


_TA = 1280
_NBLK = NA_PAD // _TA


def _amm_body(am_ref, w_ref, a_ref, ab_ref):
    am = am_ref[...].astype(jnp.bfloat16)
    acc = jnp.dot(am, w_ref[...], preferred_element_type=jnp.float32)
    ap = _pack_msg(acc)
    a_ref[...] = ap
    ab_ref[...] = ap


def _amm_call(amsg, wh_bf):
    return pl.pallas_call(
        _amm_body,
        grid=(_NBLK,),
        in_specs=[
            pl.BlockSpec((_TA, HIDDEN), lambda t: (t, 0)),
            pl.BlockSpec((HIDDEN, HIDDEN), lambda t: (0, 0)),
        ],
        out_specs=[pl.BlockSpec((_TA, 128), lambda t: (t, 0))] * 2,
        out_shape=[jax.ShapeDtypeStruct((NA_PAD, 128), jnp.int32)] * 2,
    )(amsg, wh_bf)


def _readout_body(am_ref, fa_ref, ids_ref, wo1_ref, wo2_ref, b_ref,
                  sums_ref, cnt_ref):
    t = pl.program_id(0)
    am = am_ref[...]
    hid = (jnp.dot(fa_ref[...], wo1_ref[...], preferred_element_type=jnp.float32)
           + jnp.dot(am, wo2_ref[...], preferred_element_type=jnp.float32)
           + b_ref[...])
    hid = jnp.maximum(hid, 0.0)
    ids = ids_ref[0, 0, :]
    mols = lax.broadcasted_iota(jnp.int32, (N_MOLS, _TA), 0)
    onehot = (mols == ids[None, :]).astype(jnp.float32)
    psum = jnp.dot(onehot, hid, preferred_element_type=jnp.float32)
    pcnt = jnp.broadcast_to(jnp.sum(onehot, axis=1, keepdims=True),
                            (N_MOLS, HIDDEN))

    @pl.when(t == 0)
    def _():
        sums_ref[...] = psum
        cnt_ref[...] = pcnt

    @pl.when(t > 0)
    def _():
        sums_ref[...] += psum
        cnt_ref[...] += pcnt

    @pl.when(t == _NBLK - 1)
    def _():
        s = sums_ref[...]
        c = cnt_ref[...]
        sums_ref[...] = jnp.where(c > 0.0, s / jnp.maximum(c, 1.0), 0.0)


def _readout_call(amsg, f_atoms_pad, mol3d, wo_t1, wo_t2p, wo_b2):
    return pl.pallas_call(
        _readout_body,
        grid=(_NBLK,),
        in_specs=[
            pl.BlockSpec((_TA, HIDDEN), lambda t: (t, 0)),
            pl.BlockSpec((_TA, ATOM_FDIM), lambda t: (t, 0)),
            pl.BlockSpec((1, 1, _TA), lambda t: (t, 0, 0)),
            pl.BlockSpec((ATOM_FDIM, HIDDEN), lambda t: (0, 0)),
            pl.BlockSpec((HIDDEN, HIDDEN), lambda t: (0, 0)),
            pl.BlockSpec((1, HIDDEN), lambda t: (0, 0)),
        ],
        out_specs=[
            pl.BlockSpec((N_MOLS, HIDDEN), lambda t: (0, 0)),
            pl.BlockSpec((N_MOLS, HIDDEN), lambda t: (0, 0)),
        ],
        out_shape=[
            jax.ShapeDtypeStruct((N_MOLS, HIDDEN), jnp.float32),
            jax.ShapeDtypeStruct((N_MOLS, HIDDEN), jnp.float32),
        ],
    )(amsg, f_atoms_pad, mol3d, wo_t1, wo_t2p, wo_b2)


def _softplus(x):
    return jnp.maximum(x, 0.0) + jnp.log1p(jnp.exp(-jnp.abs(x)))


def _kl_term(mu, rho):
    sig = 1e-6 + _softplus(rho)
    return jnp.sum(jnp.log(PRIOR_SIG / sig)
                   + (sig * sig + mu * mu) / (2.0 * PRIOR_SIG * PRIOR_SIG) - 0.5)


def _kl_body(wi_mu, wi_rho, wh_mu, wh_rho, wo_mu, wo_rho, wb_mu, wb_rho, o_ref):
    o_ref[0, 0] = (_kl_term(wi_mu[...], wi_rho[...])
                   + _kl_term(wh_mu[...], wh_rho[...])
                   + _kl_term(wo_mu[...], wo_rho[...])
                   + _kl_term(wb_mu[...], wb_rho[...]))


def _kl_call(wi_mu, wi_rho, wh_mu, wh_rho, wo_mu, wo_rho, wo_b, wo_b_rho):
    full = lambda s: pl.BlockSpec(s, lambda: tuple(0 for _ in s))
    out = pl.pallas_call(
        _kl_body,
        in_specs=[
            full((HIDDEN, BOND_FDIM)), full((HIDDEN, BOND_FDIM)),
            full((HIDDEN, HIDDEN)), full((HIDDEN, HIDDEN)),
            full((HIDDEN, ATOM_FDIM + HIDDEN)), full((HIDDEN, ATOM_FDIM + HIDDEN)),
            full((1, HIDDEN)), full((1, HIDDEN)),
        ],
        out_specs=pl.BlockSpec((1, 1), lambda: (0, 0), memory_space=pltpu.SMEM),
        out_shape=jax.ShapeDtypeStruct((1, 1), jnp.float32),
    )(wi_mu, wi_rho, wh_mu, wh_rho, wo_mu, wo_rho,
      wo_b.reshape(1, HIDDEN), wo_b_rho.reshape(1, HIDDEN))
    return out[0, 0]


_GA = 16
_A_PER_W = NA_PAD // NW
_GS_CHUNKS = _A_PER_W // _GA
_GS_TRIPS = (_GS_CHUNKS + 2) // 3

_sc_mesh = plsc.VectorSubcoreMesh(core_axis_name="c", subcore_axis_name="s")

def _unpack2f32(w):
    lo = lax.bitcast_convert_type(w << 16, jnp.float32)
    hi = lax.bitcast_convert_type(w & np.int32(-65536), jnp.float32)
    return lo, hi


def _gsum_kernel(msg_hbm0, msg_hbm1, msg_hbm2, msg_hbm3, idx_hbm, out_hbm,
                 idx_v, r00, r01, r02, r03, r10, r11, r12, r13,
                 r20, r21, r22, r23, a0, a1, a2,
                 g00, g01, g02, g03, g10, g11, g12, g13,
                 g20, g21, g22, g23, st0, st1, st2):
    wid = lax.axis_index("s") * _NC + lax.axis_index("c")
    rows = [[r00, r01, r02, r03], [r10, r11, r12, r13], [r20, r21, r22, r23]]
    sg = [[g00, g01, g02, g03], [g10, g11, g12, g13], [g20, g21, g22, g23]]
    accs = [a0, a1, a2]
    st = [st0, st1, st2]

    pltpu.sync_copy(idx_hbm.at[wid], idx_v)
    msgs = [msg_hbm0, msg_hbm1, msg_hbm2, msg_hbm3]

    def issue(c, s):
        for h in range(4):
            g = c * 4 + h
            pltpu.async_copy(msgs[h].at[idx_v.at[g, pl.ds(0, 32)]],
                             rows[s][h].at[pl.ds(0, 32)], sg[s][h])
            pltpu.async_copy(msgs[h].at[idx_v.at[g, pl.ds(32, 32)]],
                             rows[s][h].at[pl.ds(32, 32)], sg[s][h])

    for s in range(3):
        issue(s, s)

    def trip(t, _):
        for s in range(3):
            c = 3 * t + s

            @pl.when(c < _GS_CHUNKS)
            def _(c=c, s=s):
                for h in range(4):
                    pltpu.make_async_copy(msgs[h].at[idx_v.at[c * 4 + h]],
                                          rows[s][h], sg[s][h]).wait()


                @pl.when(c >= 3)
                def _():
                    pltpu.make_async_copy(
                        accs[s],
                        out_hbm.at[pl.ds(wid * _A_PER_W, _GA)],
                        st[s]).wait()

                for h in range(4):
                    def atom(a, _, h=h):
                        r = a * MAX_NB
                        lo = [None] * 8
                        hi = [None] * 8
                        for k in range(MAX_NB):
                            for cc in range(8):
                                l2, h2 = _unpack2f32(
                                    rows[s][h][r + k, pl.ds(cc * L, L)])
                                if k == 0:
                                    lo[cc], hi[cc] = l2, h2
                                else:
                                    lo[cc] = lo[cc] + l2
                                    hi[cc] = hi[cc] + h2
                        for cc in range(8):
                            accs[s][h * 4 + a, pl.ds(cc * L, L)] = lo[cc]
                            accs[s][h * 4 + a, pl.ds(128 + cc * L, L)] = hi[cc]
                        return 0

                    lax.fori_loop(0, 4, atom, 0)
                pltpu.async_copy(
                    accs[s],
                    out_hbm.at[pl.ds(wid * _A_PER_W + c * _GA, _GA)],
                    st[s])
                nc = c + 3

                @pl.when(nc < _GS_CHUNKS)
                def _():
                    issue(nc, s)
        return 0

    lax.fori_loop(0, _GS_TRIPS, trip, 0)
    for s in range(3):
        pltpu.make_async_copy(accs[s],
                              out_hbm.at[pl.ds(wid * _A_PER_W, _GA)],
                              st[s]).wait()


_gsum_call = functools.partial(
    pl.kernel,
    out_type=jax.ShapeDtypeStruct((NA_PAD, HIDDEN), jnp.float32),
    mesh=_sc_mesh,
    scratch_types=[
        pltpu.VMEM((_GS_CHUNKS * 4, 64), jnp.int32),
    ] + [pltpu.VMEM((64, 128), jnp.int32)] * 12
      + [pltpu.VMEM((_GA, HIDDEN), jnp.float32)] * 3
      + [pltpu.SemaphoreType.DMA] * 15,
)(_gsum_kernel)


_GB = 40
_B_PER_W = N_BONDS // NW
_CB_CHUNKS = _B_PER_W // _GB
_CB_TRIPS = (_CB_CHUNKS + 2) // 3


def _combine_kernel(a_hbm, ab_hbm, h_hbm, hb_hbm, b2a_hbm, brev_hbm, ga_hbm, gh_hbm,
                    ia_v, ir_v,
                    ra0, ra1, ra2, ra3, ra4, ra5,
                    rh0, rh1, rh2, rh3, rh4, rh5,
                    sa0, sa1, sa2, sa3, sa4, sa5,
                    sh0, sh1, sh2, sh3, sh4, sh5,
                    ta0, ta1, ta2, ta3, ta4, ta5,
                    th0, th1, th2, th3, th4, th5):
    wid = lax.axis_index("s") * _NC + lax.axis_index("c")
    ra = [ra0, ra1, ra2, ra3, ra4, ra5]
    rh = [rh0, rh1, rh2, rh3, rh4, rh5]
    sa = [sa0, sa1, sa2, sa3, sa4, sa5]
    sh = [sh0, sh1, sh2, sh3, sh4, sh5]
    ta = [ta0, ta1, ta2, ta3, ta4, ta5]
    th = [th0, th1, th2, th3, th4, th5]

    pltpu.sync_copy(b2a_hbm.at[wid], ia_v)
    pltpu.sync_copy(brev_hbm.at[wid], ir_v)

    def issue(c, s):
        pltpu.async_copy(a_hbm.at[ia_v.at[c, pl.ds(0, 24)]],
                         ra[s].at[pl.ds(0, 24)], sa[s])
        pltpu.async_copy(ab_hbm.at[ia_v.at[c, pl.ds(24, 16)]],
                         ra[s].at[pl.ds(24, 16)], sa[s])
        pltpu.async_copy(h_hbm.at[ir_v.at[c, pl.ds(0, 24)]],
                         rh[s].at[pl.ds(0, 24)], sh[s])
        pltpu.async_copy(hb_hbm.at[ir_v.at[c, pl.ds(24, 16)]],
                         rh[s].at[pl.ds(24, 16)], sh[s])

    for s in range(3):
        issue(s, s)

    def out_sl(c):
        return pl.ds(wid * _B_PER_W + c * _GB, _GB)

    def trip(t, _):
        for s in range(6):
            c = 6 * t + s

            @pl.when(c < _CB_CHUNKS)
            def _(c=c, s=s):
                pltpu.make_async_copy(a_hbm.at[ia_v.at[c]], ra[s], sa[s]).wait()
                pltpu.make_async_copy(h_hbm.at[ir_v.at[c]], rh[s], sh[s]).wait()
                pltpu.async_copy(ra[s], ga_hbm.at[out_sl(c)], ta[s])
                pltpu.async_copy(rh[s], gh_hbm.at[out_sl(c)], th[s])
                nc = c + 3
                s2 = (s + 3) % 6

                @pl.when(nc < _CB_CHUNKS)
                def _():
                    @pl.when(c >= 3)
                    def _():
                        pltpu.make_async_copy(ra[s2], ga_hbm.at[out_sl(c)],
                                              ta[s2]).wait()
                        pltpu.make_async_copy(rh[s2], gh_hbm.at[out_sl(c)],
                                              th[s2]).wait()
                    issue(nc, s2)
        return 0

    lax.fori_loop(0, (_CB_CHUNKS + 5) // 6, trip, 0)
    for s in range(6):
        pltpu.make_async_copy(ra[s], ga_hbm.at[out_sl(0)], ta[s]).wait()
        pltpu.make_async_copy(rh[s], gh_hbm.at[out_sl(0)], th[s]).wait()


_combine_call = functools.partial(
    pl.kernel,
    out_type=[
        jax.ShapeDtypeStruct((N_BONDS, 128), jnp.int32),
        jax.ShapeDtypeStruct((N_BONDS, 128), jnp.int32),
    ],
    mesh=_sc_mesh,
    scratch_types=[
        pltpu.VMEM((_CB_CHUNKS, _GB), jnp.int32),
        pltpu.VMEM((_CB_CHUNKS, _GB), jnp.int32),
    ] + [pltpu.VMEM((_GB, 128), jnp.int32)] * 12
      + [pltpu.SemaphoreType.DMA] * 24,
)(_combine_kernel)


def kernel(f_atoms, f_bonds, a2b, b2a, b2revb, mol_ids,
           Wi_mu, Wi_rho, Wh_mu, Wh_rho, Wo_mu, Wo_rho, Wo_b, Wo_b_rho):
    wi_t = Wi_mu.T
    wh_bf = Wh_mu.T.astype(jnp.bfloat16)
    wo_t1 = Wo_mu.T[:ATOM_FDIM, :]
    wo_t2p = Wo_mu.T[ATOM_FDIM:, :]
    wo_b2 = Wo_b.reshape(1, HIDDEN)

    pad_a = NA_PAD - N_ATOMS
    f_atoms_pad = jnp.pad(f_atoms, ((0, pad_a), (0, 0)))
    a2b_w = jnp.pad(a2b, ((0, pad_a), (0, 0))).reshape(
        NW, _GS_CHUNKS * 4, 64)
    b2a_w = b2a.reshape(NW, _CB_CHUNKS, _GB)
    brev_w = b2revb.reshape(NW, _CB_CHUNKS, _GB)
    mol3d = jnp.pad(mol_ids, (0, pad_a), constant_values=N_MOLS).reshape(
        _NBLK, 1, _TA)

    inp, msg3, msg3b, msg3c, msg3d = _wi_call(f_bonds, wi_t)
    tkl = _kl_call(Wi_mu, Wi_rho, Wh_mu, Wh_rho, Wo_mu, Wo_rho, Wo_b, Wo_b_rho)

    amsg = _gsum_call(msg3, msg3b, msg3c, msg3d, a2b_w)
    h_mat, h_matb = _mm_call(msg3, wh_bf)
    a_mat, a_matb = _amm_call(amsg, wh_bf)

    outs = []
    for d in range(1, DEPTH_MAX):
        ga, gh = _combine_call(a_mat, a_matb, h_mat, h_matb, b2a_w, brev_w)
        if d < DEPTH_MAX - 1:
            msg3, msg3b, msg3c, msg3d, h_mat, h_matb = _relu_mm_call(ga, gh, inp, wh_bf)
        else:
            msg3, msg3b, msg3c, msg3d = _relu_call(ga, gh, inp)
        amsg = _gsum_call(msg3, msg3b, msg3c, msg3d, a2b_w)
        a_mat, a_matb = _amm_call(amsg, wh_bf)
        mol_vecs, _ = _readout_call(amsg, f_atoms_pad, mol3d,
                                    wo_t1, wo_t2p, wo_b2)
        outs.append(mol_vecs)

    return tuple(outs) + (tkl,)


# speedup vs baseline: 1.0389x; 1.0389x over previous
import functools

import numpy as np
import jax
import jax.numpy as jnp
from jax import lax
from jax.experimental import pallas as pl
from jax.experimental.pallas import tpu as pltpu
from jax.experimental.pallas import tpu_sc as plsc

N_ATOMS = 10000
N_BONDS = 160000
MAX_NB = 16
ATOM_FDIM = 128
BOND_FDIM = 144
HIDDEN = 256
N_MOLS = 100
PRIOR_SIG = 0.1
DEPTH_MAX = 5

NA_PAD = 10240
L = 16

_info = plsc.get_sparse_core_info()
_NC, _NS = _info.num_cores, _info.num_subcores
NW = _NC * _NS

_BM = 1600


def _pack_msg(x):
    xb = x.astype(jnp.bfloat16)
    lo = lax.bitcast_convert_type(xb[:, :128], jnp.uint16).astype(jnp.int32)
    hi = lax.bitcast_convert_type(xb[:, 128:], jnp.uint16).astype(jnp.int32)
    return lo | (hi << 16)


def _unpack_msg_bf(w):
    lo = lax.bitcast_convert_type((w & 0xFFFF).astype(jnp.uint16), jnp.bfloat16)
    hi = lax.bitcast_convert_type(
        lax.shift_right_logical(w, 16).astype(jnp.uint16), jnp.bfloat16)
    return jnp.concatenate([lo, hi], axis=1)


def _wi_body(x_ref, w_ref, inp_ref, msg_ref, msgb_ref, msgc_ref, msgd_ref):
    acc = jnp.dot(x_ref[...], w_ref[...], preferred_element_type=jnp.float32)
    inp_ref[...] = _pack_msg(acc)
    m = _pack_msg(jnp.maximum(acc, 0.0))
    msg_ref[...] = m
    msgb_ref[...] = m
    msgc_ref[...] = m
    msgd_ref[...] = m


def _wi_call(f_bonds, wi_t):
    grid = N_BONDS // _BM
    return pl.pallas_call(
        _wi_body,
        grid=(grid,),
        in_specs=[
            pl.BlockSpec((_BM, BOND_FDIM), lambda t: (t, 0)),
            pl.BlockSpec((BOND_FDIM, HIDDEN), lambda t: (0, 0)),
        ],
        out_specs=[pl.BlockSpec((_BM, 128), lambda t: (t, 0))] * 5,
        out_shape=[jax.ShapeDtypeStruct((N_BONDS, 128), jnp.int32)] * 5,
    )(f_bonds, wi_t)


def _mm_body(m_ref, w_ref, h_ref):
    m = _unpack_msg_bf(m_ref[...])
    acc = jnp.dot(m, w_ref[...], preferred_element_type=jnp.float32)
    h_ref[...] = _pack_msg(acc)


def _mm_call(msg3, wh_bf):
    grid = N_BONDS // _BM
    return pl.pallas_call(
        _mm_body,
        grid=(grid,),
        in_specs=[
            pl.BlockSpec((_BM, 128), lambda t: (t, 0)),
            pl.BlockSpec((HIDDEN, HIDDEN), lambda t: (0, 0)),
        ],
        out_specs=pl.BlockSpec((_BM, 128), lambda t: (t, 0)),
        out_shape=jax.ShapeDtypeStruct((N_BONDS, 128), jnp.int32),
    )(msg3, wh_bf)


def _relu_mm_body(ga_ref, gh_ref, inp_ref, w_ref, msg_ref, msgb_ref, msgc_ref, msgd_ref, h_ref):
    a = _unpack_msg_bf(ga_ref[...]).astype(jnp.float32)
    h = _unpack_msg_bf(gh_ref[...]).astype(jnp.float32)
    i = _unpack_msg_bf(inp_ref[...]).astype(jnp.float32)
    m = jnp.maximum(i + (a - h), 0.0)
    mp = _pack_msg(m)
    msg_ref[...] = mp
    msgb_ref[...] = mp
    msgc_ref[...] = mp
    msgd_ref[...] = mp
    acc = jnp.dot(m.astype(jnp.bfloat16), w_ref[...],
                  preferred_element_type=jnp.float32)
    h_ref[...] = _pack_msg(acc)


def _relu_mm_call(ga, gh, inp, wh_bf):
    grid = N_BONDS // _BM
    return pl.pallas_call(
        _relu_mm_body,
        grid=(grid,),
        in_specs=[
            pl.BlockSpec((_BM, 128), lambda t: (t, 0)),
            pl.BlockSpec((_BM, 128), lambda t: (t, 0)),
            pl.BlockSpec((_BM, 128), lambda t: (t, 0)),
            pl.BlockSpec((HIDDEN, HIDDEN), lambda t: (0, 0)),
        ],
        out_specs=[pl.BlockSpec((_BM, 128), lambda t: (t, 0))] * 5,
        out_shape=[jax.ShapeDtypeStruct((N_BONDS, 128), jnp.int32)] * 5,
    )(ga, gh, inp, wh_bf)


def _relu_body(ga_ref, gh_ref, inp_ref, msg_ref, msgb_ref, msgc_ref, msgd_ref):
    a = _unpack_msg_bf(ga_ref[...]).astype(jnp.float32)
    h = _unpack_msg_bf(gh_ref[...]).astype(jnp.float32)
    i = _unpack_msg_bf(inp_ref[...]).astype(jnp.float32)
    m = _pack_msg(jnp.maximum(i + (a - h), 0.0))
    msg_ref[...] = m
    msgb_ref[...] = m
    msgc_ref[...] = m
    msgd_ref[...] = m


def _relu_call(ga, gh, inp):
    grid = N_BONDS // _BM
    return pl.pallas_call(
        _relu_body,
        grid=(grid,),
        in_specs=[
            pl.BlockSpec((_BM, 128), lambda t: (t, 0)),
            pl.BlockSpec((_BM, 128), lambda t: (t, 0)),
            pl.BlockSpec((_BM, 128), lambda t: (t, 0)),
        ],
        out_specs=[pl.BlockSpec((_BM, 128), lambda t: (t, 0))] * 4,
        out_shape=[jax.ShapeDtypeStruct((N_BONDS, 128), jnp.int32)] * 4,
    )(ga, gh, inp)


_TA = 1280
_NBLK = NA_PAD // _TA


def _amm_body(am_ref, w_ref, a_ref):
    am = am_ref[...].astype(jnp.bfloat16)
    acc = jnp.dot(am, w_ref[...], preferred_element_type=jnp.float32)
    a_ref[...] = _pack_msg(acc)


def _amm_call(amsg, wh_bf):
    return pl.pallas_call(
        _amm_body,
        grid=(_NBLK,),
        in_specs=[
            pl.BlockSpec((_TA, HIDDEN), lambda t: (t, 0)),
            pl.BlockSpec((HIDDEN, HIDDEN), lambda t: (0, 0)),
        ],
        out_specs=pl.BlockSpec((_TA, 128), lambda t: (t, 0)),
        out_shape=jax.ShapeDtypeStruct((NA_PAD, 128), jnp.int32),
    )(amsg, wh_bf)


def _readout_body(am_ref, fa_ref, ids_ref, wo1_ref, wo2_ref, b_ref,
                  sums_ref, cnt_ref):
    t = pl.program_id(0)
    am = am_ref[...]
    hid = (jnp.dot(fa_ref[...], wo1_ref[...], preferred_element_type=jnp.float32)
           + jnp.dot(am, wo2_ref[...], preferred_element_type=jnp.float32)
           + b_ref[...])
    hid = jnp.maximum(hid, 0.0)
    ids = ids_ref[0, 0, :]
    mols = lax.broadcasted_iota(jnp.int32, (N_MOLS, _TA), 0)
    onehot = (mols == ids[None, :]).astype(jnp.float32)
    psum = jnp.dot(onehot, hid, preferred_element_type=jnp.float32)
    pcnt = jnp.broadcast_to(jnp.sum(onehot, axis=1, keepdims=True),
                            (N_MOLS, HIDDEN))

    @pl.when(t == 0)
    def _():
        sums_ref[...] = psum
        cnt_ref[...] = pcnt

    @pl.when(t > 0)
    def _():
        sums_ref[...] += psum
        cnt_ref[...] += pcnt

    @pl.when(t == _NBLK - 1)
    def _():
        s = sums_ref[...]
        c = cnt_ref[...]
        sums_ref[...] = jnp.where(c > 0.0, s / jnp.maximum(c, 1.0), 0.0)


def _readout_call(amsg, f_atoms_pad, mol3d, wo_t1, wo_t2p, wo_b2):
    return pl.pallas_call(
        _readout_body,
        grid=(_NBLK,),
        in_specs=[
            pl.BlockSpec((_TA, HIDDEN), lambda t: (t, 0)),
            pl.BlockSpec((_TA, ATOM_FDIM), lambda t: (t, 0)),
            pl.BlockSpec((1, 1, _TA), lambda t: (t, 0, 0)),
            pl.BlockSpec((ATOM_FDIM, HIDDEN), lambda t: (0, 0)),
            pl.BlockSpec((HIDDEN, HIDDEN), lambda t: (0, 0)),
            pl.BlockSpec((1, HIDDEN), lambda t: (0, 0)),
        ],
        out_specs=[
            pl.BlockSpec((N_MOLS, HIDDEN), lambda t: (0, 0)),
            pl.BlockSpec((N_MOLS, HIDDEN), lambda t: (0, 0)),
        ],
        out_shape=[
            jax.ShapeDtypeStruct((N_MOLS, HIDDEN), jnp.float32),
            jax.ShapeDtypeStruct((N_MOLS, HIDDEN), jnp.float32),
        ],
    )(amsg, f_atoms_pad, mol3d, wo_t1, wo_t2p, wo_b2)


def _softplus(x):
    return jnp.maximum(x, 0.0) + jnp.log1p(jnp.exp(-jnp.abs(x)))


def _kl_term(mu, rho):
    sig = 1e-6 + _softplus(rho)
    return jnp.sum(jnp.log(PRIOR_SIG / sig)
                   + (sig * sig + mu * mu) / (2.0 * PRIOR_SIG * PRIOR_SIG) - 0.5)


def _kl_body(wi_mu, wi_rho, wh_mu, wh_rho, wo_mu, wo_rho, wb_mu, wb_rho, o_ref):
    o_ref[0, 0] = (_kl_term(wi_mu[...], wi_rho[...])
                   + _kl_term(wh_mu[...], wh_rho[...])
                   + _kl_term(wo_mu[...], wo_rho[...])
                   + _kl_term(wb_mu[...], wb_rho[...]))


def _kl_call(wi_mu, wi_rho, wh_mu, wh_rho, wo_mu, wo_rho, wo_b, wo_b_rho):
    full = lambda s: pl.BlockSpec(s, lambda: tuple(0 for _ in s))
    out = pl.pallas_call(
        _kl_body,
        in_specs=[
            full((HIDDEN, BOND_FDIM)), full((HIDDEN, BOND_FDIM)),
            full((HIDDEN, HIDDEN)), full((HIDDEN, HIDDEN)),
            full((HIDDEN, ATOM_FDIM + HIDDEN)), full((HIDDEN, ATOM_FDIM + HIDDEN)),
            full((1, HIDDEN)), full((1, HIDDEN)),
        ],
        out_specs=pl.BlockSpec((1, 1), lambda: (0, 0), memory_space=pltpu.SMEM),
        out_shape=jax.ShapeDtypeStruct((1, 1), jnp.float32),
    )(wi_mu, wi_rho, wh_mu, wh_rho, wo_mu, wo_rho,
      wo_b.reshape(1, HIDDEN), wo_b_rho.reshape(1, HIDDEN))
    return out[0, 0]


_GA = 16
_A_PER_W = NA_PAD // NW
_GS_CHUNKS = _A_PER_W // _GA
_GS_TRIPS = (_GS_CHUNKS + 2) // 3

_sc_mesh = plsc.VectorSubcoreMesh(core_axis_name="c", subcore_axis_name="s")

def _unpack2f32(w):
    lo = lax.bitcast_convert_type(w << 16, jnp.float32)
    hi = lax.bitcast_convert_type(w & np.int32(-65536), jnp.float32)
    return lo, hi


def _gsum_kernel(msg_hbm0, msg_hbm1, msg_hbm2, msg_hbm3, idx_hbm, out_hbm,
                 idx_v, r00, r01, r02, r03, r10, r11, r12, r13,
                 r20, r21, r22, r23, a0, a1, a2,
                 g00, g01, g02, g03, g10, g11, g12, g13,
                 g20, g21, g22, g23, st0, st1, st2):
    wid = lax.axis_index("s") * _NC + lax.axis_index("c")
    rows = [[r00, r01, r02, r03], [r10, r11, r12, r13], [r20, r21, r22, r23]]
    sg = [[g00, g01, g02, g03], [g10, g11, g12, g13], [g20, g21, g22, g23]]
    accs = [a0, a1, a2]
    st = [st0, st1, st2]

    pltpu.sync_copy(idx_hbm.at[wid], idx_v)
    msgs = [msg_hbm0, msg_hbm1, msg_hbm2, msg_hbm3]

    def issue(c, s):
        for h in range(4):
            g = c * 4 + h
            pltpu.async_copy(msgs[h].at[idx_v.at[g, pl.ds(0, 32)]],
                             rows[s][h].at[pl.ds(0, 32)], sg[s][h])
            pltpu.async_copy(msgs[h].at[idx_v.at[g, pl.ds(32, 32)]],
                             rows[s][h].at[pl.ds(32, 32)], sg[s][h])

    for s in range(3):
        issue(s, s)

    def trip(t, _):
        for s in range(3):
            c = 3 * t + s

            @pl.when(c < _GS_CHUNKS)
            def _(c=c, s=s):
                for h in range(4):
                    pltpu.make_async_copy(msgs[h].at[idx_v.at[c * 4 + h]],
                                          rows[s][h], sg[s][h]).wait()


                @pl.when(c >= 3)
                def _():
                    pltpu.make_async_copy(
                        accs[s],
                        out_hbm.at[pl.ds(wid * _A_PER_W, _GA)],
                        st[s]).wait()

                for h in range(4):
                    def atom(a, _, h=h):
                        r = a * MAX_NB
                        lo = [None] * 8
                        hi = [None] * 8
                        for k in range(MAX_NB):
                            for cc in range(8):
                                l2, h2 = _unpack2f32(
                                    rows[s][h][r + k, pl.ds(cc * L, L)])
                                if k == 0:
                                    lo[cc], hi[cc] = l2, h2
                                else:
                                    lo[cc] = lo[cc] + l2
                                    hi[cc] = hi[cc] + h2
                        for cc in range(8):
                            accs[s][h * 4 + a, pl.ds(cc * L, L)] = lo[cc]
                            accs[s][h * 4 + a, pl.ds(128 + cc * L, L)] = hi[cc]
                        return 0

                    lax.fori_loop(0, 4, atom, 0)
                pltpu.async_copy(
                    accs[s],
                    out_hbm.at[pl.ds(wid * _A_PER_W + c * _GA, _GA)],
                    st[s])
                nc = c + 3

                @pl.when(nc < _GS_CHUNKS)
                def _():
                    issue(nc, s)
        return 0

    lax.fori_loop(0, _GS_TRIPS, trip, 0)
    for s in range(3):
        pltpu.make_async_copy(accs[s],
                              out_hbm.at[pl.ds(wid * _A_PER_W, _GA)],
                              st[s]).wait()


_gsum_call = functools.partial(
    pl.kernel,
    out_type=jax.ShapeDtypeStruct((NA_PAD, HIDDEN), jnp.float32),
    mesh=_sc_mesh,
    scratch_types=[
        pltpu.VMEM((_GS_CHUNKS * 4, 64), jnp.int32),
    ] + [pltpu.VMEM((64, 128), jnp.int32)] * 12
      + [pltpu.VMEM((_GA, HIDDEN), jnp.float32)] * 3
      + [pltpu.SemaphoreType.DMA] * 15,
)(_gsum_kernel)


_GB = 40
_B_PER_W = N_BONDS // NW
_CB_CHUNKS = _B_PER_W // _GB
_CB_TRIPS = (_CB_CHUNKS + 2) // 3


def _combine_kernel(a_hbm, h_hbm, b2a_hbm, brev_hbm, ga_hbm, gh_hbm,
                    ia_v, ir_v,
                    ra0, ra1, ra2, ra3, ra4, ra5,
                    rh0, rh1, rh2, rh3, rh4, rh5,
                    sa0, sa1, sa2, sa3, sa4, sa5,
                    sh0, sh1, sh2, sh3, sh4, sh5,
                    ta0, ta1, ta2, ta3, ta4, ta5,
                    th0, th1, th2, th3, th4, th5):
    wid = lax.axis_index("s") * _NC + lax.axis_index("c")
    ra = [ra0, ra1, ra2, ra3, ra4, ra5]
    rh = [rh0, rh1, rh2, rh3, rh4, rh5]
    sa = [sa0, sa1, sa2, sa3, sa4, sa5]
    sh = [sh0, sh1, sh2, sh3, sh4, sh5]
    ta = [ta0, ta1, ta2, ta3, ta4, ta5]
    th = [th0, th1, th2, th3, th4, th5]

    pltpu.sync_copy(b2a_hbm.at[wid], ia_v)
    pltpu.sync_copy(brev_hbm.at[wid], ir_v)

    def issue(c, s):
        pltpu.async_copy(a_hbm.at[ia_v.at[c]], ra[s], sa[s])
        pltpu.async_copy(h_hbm.at[ir_v.at[c]], rh[s], sh[s])

    for s in range(3):
        issue(s, s)

    def out_sl(c):
        return pl.ds(wid * _B_PER_W + c * _GB, _GB)

    def trip(t, _):
        for s in range(6):
            c = 6 * t + s

            @pl.when(c < _CB_CHUNKS)
            def _(c=c, s=s):
                pltpu.make_async_copy(a_hbm.at[ia_v.at[c]], ra[s], sa[s]).wait()
                pltpu.make_async_copy(h_hbm.at[ir_v.at[c]], rh[s], sh[s]).wait()
                pltpu.async_copy(ra[s], ga_hbm.at[out_sl(c)], ta[s])
                pltpu.async_copy(rh[s], gh_hbm.at[out_sl(c)], th[s])
                nc = c + 3
                s2 = (s + 3) % 6

                @pl.when(nc < _CB_CHUNKS)
                def _():
                    @pl.when(c >= 3)
                    def _():
                        pltpu.make_async_copy(ra[s2], ga_hbm.at[out_sl(c)],
                                              ta[s2]).wait()
                        pltpu.make_async_copy(rh[s2], gh_hbm.at[out_sl(c)],
                                              th[s2]).wait()
                    issue(nc, s2)
        return 0

    lax.fori_loop(0, (_CB_CHUNKS + 5) // 6, trip, 0)
    for s in range(6):
        pltpu.make_async_copy(ra[s], ga_hbm.at[out_sl(0)], ta[s]).wait()
        pltpu.make_async_copy(rh[s], gh_hbm.at[out_sl(0)], th[s]).wait()


_combine_call = functools.partial(
    pl.kernel,
    out_type=[
        jax.ShapeDtypeStruct((N_BONDS, 128), jnp.int32),
        jax.ShapeDtypeStruct((N_BONDS, 128), jnp.int32),
    ],
    mesh=_sc_mesh,
    scratch_types=[
        pltpu.VMEM((_CB_CHUNKS, _GB), jnp.int32),
        pltpu.VMEM((_CB_CHUNKS, _GB), jnp.int32),
    ] + [pltpu.VMEM((_GB, 128), jnp.int32)] * 12
      + [pltpu.SemaphoreType.DMA] * 24,
)(_combine_kernel)


def kernel(f_atoms, f_bonds, a2b, b2a, b2revb, mol_ids,
           Wi_mu, Wi_rho, Wh_mu, Wh_rho, Wo_mu, Wo_rho, Wo_b, Wo_b_rho):
    wi_t = Wi_mu.T
    wh_bf = Wh_mu.T.astype(jnp.bfloat16)
    wo_t1 = Wo_mu.T[:ATOM_FDIM, :]
    wo_t2p = Wo_mu.T[ATOM_FDIM:, :]
    wo_b2 = Wo_b.reshape(1, HIDDEN)

    pad_a = NA_PAD - N_ATOMS
    f_atoms_pad = jnp.pad(f_atoms, ((0, pad_a), (0, 0)))
    a2b_w = jnp.pad(a2b, ((0, pad_a), (0, 0))).reshape(
        NW, _GS_CHUNKS * 4, 64)
    b2a_w = b2a.reshape(NW, _CB_CHUNKS, _GB)
    brev_w = b2revb.reshape(NW, _CB_CHUNKS, _GB)
    mol3d = jnp.pad(mol_ids, (0, pad_a), constant_values=N_MOLS).reshape(
        _NBLK, 1, _TA)

    inp, msg3, msg3b, msg3c, msg3d = _wi_call(f_bonds, wi_t)
    tkl = _kl_call(Wi_mu, Wi_rho, Wh_mu, Wh_rho, Wo_mu, Wo_rho, Wo_b, Wo_b_rho)

    amsg = _gsum_call(msg3, msg3b, msg3c, msg3d, a2b_w)
    h_mat = _mm_call(msg3, wh_bf)
    a_mat = _amm_call(amsg, wh_bf)

    outs = []
    for d in range(1, DEPTH_MAX):
        ga, gh = _combine_call(a_mat, h_mat, b2a_w, brev_w)
        if d < DEPTH_MAX - 1:
            msg3, msg3b, msg3c, msg3d, h_mat = _relu_mm_call(ga, gh, inp, wh_bf)
        else:
            msg3, msg3b, msg3c, msg3d = _relu_call(ga, gh, inp)
        amsg = _gsum_call(msg3, msg3b, msg3c, msg3d, a2b_w)
        a_mat = _amm_call(amsg, wh_bf)
        mol_vecs, _ = _readout_call(amsg, f_atoms_pad, mol3d,
                                    wo_t1, wo_t2p, wo_b2)
        outs.append(mol_vecs)

    return tuple(outs) + (tkl,)
